# baseline (speedup 1.0000x reference)
import functools

import jax
import jax.numpy as jnp
from jax import lax
from jax.experimental import pallas as pl
from jax.experimental.pallas import tpu as pltpu
from jax.experimental.pallas import tpu_sc as plsc

NUM_USERS = 1000000
NUM_ITEMS = 1000000
EMBED_DIM = 64
BATCH = 16384

_info = plsc.get_sparse_core_info()
NC = _info.num_cores
NS = _info.num_subcores
L = _info.num_lanes
NW = NC * NS
BPW = BATCH // NW

PASS_COLS = 256
NPASS = (NUM_USERS + PASS_COLS - 1) // PASS_COLS
LAST_LO = ((NUM_USERS - PASS_COLS + 127) // 128) * 128
LCAP = 2048
MCAP = 256
DUMP = BATCH
IDVECS = BATCH // L

_mesh = plsc.VectorSubcoreMesh(core_axis_name="c", subcore_axis_name="s")
_params = pltpu.CompilerParams(needs_layout_passes=False)


def _helpers():
    lane = lax.iota(jnp.int32, L)
    perms = [lane ^ m for m in (1, 2, 4, 8)]
    rowidx = [lane + L * kk for kk in range(EMBED_DIM // L)]
    jconst = [jnp.full((L,), j, jnp.int32) for j in range(L)]
    dnums = lax.GatherDimensionNumbers(
        offset_dims=(), collapsed_slice_dims=(0,), start_index_map=(0,))

    def shuffle(x, idx):
        return lax.gather(x, idx[:, None], dnums, (1,),
                          mode=lax.GatherScatterMode.PROMISE_IN_BOUNDS)

    def lanesum(s):
        for p in perms:
            s = s + shuffle(s, p)
        return s

    return lane, rowidx, jconst, shuffle, lanesum


def _make_extract():
    @functools.partial(
        pl.kernel,
        mesh=_mesh,
        out_type=[
            jax.ShapeDtypeStruct((BATCH + 1, 1, EMBED_DIM), jnp.float32),
            jax.ShapeDtypeStruct((BATCH + 1, 1, EMBED_DIM), jnp.float32),
        ],
        compiler_params=_params,
        scratch_types=[
            pltpu.VMEM((BATCH,), jnp.int32),
            pltpu.VMEM((LCAP + L,), jnp.int32),
            pltpu.VMEM((LCAP + L,), jnp.int32),
            pltpu.VMEM((LCAP + L,), jnp.int32),
            pltpu.VMEM((LCAP + L,), jnp.int32),
            pltpu.VMEM((MCAP + L,), jnp.int32),
            pltpu.VMEM((MCAP + L,), jnp.int32),
            pltpu.VMEM((2, EMBED_DIM, PASS_COLS), jnp.float32),
            pltpu.VMEM((2, EMBED_DIM, PASS_COLS), jnp.float32),
            pltpu.VMEM((4, 1, EMBED_DIM), jnp.float32),
            [pltpu.SemaphoreType.DMA] * 2,
            [pltpu.SemaphoreType.DMA] * 2,
            [pltpu.SemaphoreType.DMA] * 4,
        ],
    )
    def k1(ut_hbm, it_hbm, uid_hbm, iid_hbm, ustage, istage,
           ids_v, lidu, lposu, lidi, lposi, mid_v, mpos_v,
           pbu, pbi, rowst, sems_pu, sems_pi, sems_r):
        lane, rowidx, jconst, shuffle, _ = _helpers()
        wid = lax.axis_index("s") * NC + lax.axis_index("c")
        p0 = (NPASS * wid) // NW
        p1 = (NPASS * (wid + 1)) // NW
        npass = p1 - p0

        def filt(ids_hbm, lid, lpos):
            pltpu.sync_copy(ids_hbm, ids_v)

            def body(kk, ptr):
                vec = ids_v[pl.ds(kk * L, L)]
                pv = vec >> 8
                m = (pv >= p0) & (pv < p1)
                pos = lane + kk * L
                plsc.store_compressed(lid.at[pl.ds(ptr, L)], vec, mask=m)
                plsc.store_compressed(lpos.at[pl.ds(ptr, L)], pos, mask=m)
                cnt = plsc.all_reduce_population_count(m)[0]
                return jnp.minimum(ptr + cnt, LCAP)

            return lax.fori_loop(0, IDVECS, body, 0)

        ucnt = filt(uid_hbm, lidu, lposu)
        icnt = filt(iid_hbm, lidi, lposi)

        for s in range(4):
            pltpu.async_copy(rowst.at[s], ustage.at[DUMP], sems_r[s])

        def issue_pass(t, slot):
            lo = pl.multiple_of(
                jnp.minimum((p0 + t) * PASS_COLS, LAST_LO), 128)
            pltpu.async_copy(ut_hbm.at[:, pl.ds(lo, PASS_COLS)],
                             pbu.at[slot], sems_pu[slot])
            pltpu.async_copy(it_hbm.at[:, pl.ds(lo, PASS_COLS)],
                             pbi.at[slot], sems_pi[slot])

        issue_pass(0, 0)
        issue_pass(1, 1)

        def process(t, slot):
            pltpu.make_async_copy(ut_hbm.at[:, pl.ds(0, PASS_COLS)],
                                  pbu.at[slot], sems_pu[slot]).wait()
            pltpu.make_async_copy(it_hbm.at[:, pl.ds(0, PASS_COLS)],
                                  pbi.at[slot], sems_pi[slot]).wait()
            ps = p0 + t
            lo = jnp.minimum(ps * PASS_COLS, LAST_LO)

            for lid, lpos, lcnt, pbuf, stage in (
                    (lidu, lposu, ucnt, pbu, ustage),
                    (lidi, lposi, icnt, pbi, istage)):

                def scan(kk, mp):
                    vec = lid[pl.ds(kk * L, L)]
                    m = ((vec >> 8) == ps) & (kk * L + lane < lcnt)
                    plsc.store_compressed(mid_v.at[pl.ds(mp, L)], vec, mask=m)
                    plsc.store_compressed(
                        mpos_v.at[pl.ds(mp, L)], lpos[pl.ds(kk * L, L)],
                        mask=m)
                    cnt = plsc.all_reduce_population_count(m)[0]
                    return jnp.minimum(mp + cnt, MCAP)

                mcnt = lax.fori_loop(0, (lcnt + L - 1) // L, scan, 0)

                def ext(kk, carry):
                    mv = mid_v[pl.ds(kk * L, L)]
                    pvv = mpos_v[pl.ds(kk * L, L)]
                    colv = jnp.clip(mv - lo, 0, PASS_COLS - 1)
                    for j in range(L):
                        sr = j % 4
                        cj = shuffle(colv, jconst[j])
                        active = kk * L + j < mcnt
                        pos = jnp.where(active, pvv[j], DUMP)
                        pltpu.make_async_copy(stage.at[DUMP],
                                              rowst.at[sr], sems_r[sr]).wait()
                        for q in range(EMBED_DIM // L):
                            g = plsc.load_gather(pbuf.at[slot],
                                                 [rowidx[q], cj])
                            rowst[sr, 0, pl.ds(q * L, L)] = g
                        pltpu.async_copy(rowst.at[sr], stage.at[pos],
                                         sems_r[sr])
                    return carry

                lax.fori_loop(0, (mcnt + L - 1) // L, ext, 0)

            issue_pass(t + 2, slot)

        def pair(m, carry):
            process(2 * m, 0)
            process(2 * m + 1, 1)
            return carry

        lax.fori_loop(0, (npass + 1) // 2, pair, 0)

        for slot in range(2):
            pltpu.make_async_copy(ut_hbm.at[:, pl.ds(0, PASS_COLS)],
                                  pbu.at[slot], sems_pu[slot]).wait()
            pltpu.make_async_copy(it_hbm.at[:, pl.ds(0, PASS_COLS)],
                                  pbi.at[slot], sems_pi[slot]).wait()
        for s in range(4):
            pltpu.make_async_copy(ustage.at[DUMP],
                                  rowst.at[s], sems_r[s]).wait()

    return k1


def _make_dot():
    CH = 128

    @functools.partial(
        pl.kernel,
        mesh=_mesh,
        out_type=jax.ShapeDtypeStruct((BATCH,), jnp.float32),
        compiler_params=_params,
        scratch_types=[
            pltpu.VMEM((CH, 1, EMBED_DIM), jnp.float32),
            pltpu.VMEM((CH, 1, EMBED_DIM), jnp.float32),
            pltpu.VMEM((BPW,), jnp.float32),
        ],
    )
    def k2(ustage, istage, out_hbm, uch, ich, out_v):
        lane, _, _, _, lanesum = _helpers()
        wid = lax.axis_index("s") * NC + lax.axis_index("c")
        base = wid * BPW

        def chunk(c, carry):
            pltpu.sync_copy(ustage.at[pl.ds(base + c * CH, CH)], uch)
            pltpu.sync_copy(istage.at[pl.ds(base + c * CH, CH)], ich)

            def group(g, carry2):
                res = jnp.zeros((L,), jnp.float32)
                for r in range(L):
                    row = g * L + r
                    s = None
                    for q in range(EMBED_DIM // L):
                        uu = uch[row, 0, pl.ds(q * L, L)]
                        ii = ich[row, 0, pl.ds(q * L, L)]
                        s = uu * ii if s is None else s + uu * ii
                    res = jnp.where(lane == r, lanesum(s), res)
                y = 1.0 / (1.0 + jnp.exp(-res))
                out_v[pl.ds(c * CH + g * L, L)] = y
                return carry2

            lax.fori_loop(0, CH // L, group, 0)
            return carry

        lax.fori_loop(0, BPW // CH, chunk, 0)
        pltpu.sync_copy(out_v, out_hbm.at[pl.ds(base, BPW)])

    return k2


_extract_call = _make_extract()
_dot_call = _make_dot()


def kernel(user_emb, item_emb, user_id, item_id):
    uid = jnp.asarray(user_id, jnp.int32)
    iid = jnp.asarray(item_id, jnp.int32)
    ustage, istage = _extract_call(user_emb.T, item_emb.T, uid, iid)
    return _dot_call(ustage, istage)
